# baseline (speedup 1.0000x reference)
import jax
import jax.numpy as jnp
from jax.experimental import pallas as pl
from jax.experimental.pallas import tpu as pltpu

DM = 64
N = 4096
KC = 32
MA = 256
ALPHA = 0.98


def _final_body(stz_ref, ni_ref, ip_ref, pn_ref, ps_ref, po_ref,
                ns_out, nip_out, loss_out, acc_ref):
    i = pl.program_id(0)
    ni = ni_ref[...]
    acc = jnp.zeros((DM, DM), jnp.float32)
    nip = jnp.zeros_like(ni)
    for d in range(DM):
        nip = nip + ni[:, d:d + 1] * pn_ref[:, d, :]
    diff = ip_ref[...] - ni
    part = jnp.sum(0.5 * diff * diff)

    @pl.when(i == 0)
    def _():
        acc_ref[0] = 0.0

    acc_ref[0] += part

    @pl.when(i == pl.num_programs(0) - 1)
    def _():
        loss_out[...] = jnp.full((1, 1), acc_ref[0] / (N * DM), jnp.float32)

    nip_out[...] = nip
    ns_out[...] = (stz_ref[...] + ni + nip * ps_ref[...] + po_ref[...]) * ALPHA


def kernel(node_state, node_exhaustion, input_pred, act_noise, nodes,
           pred_nodes, nodes_act, pred_scale, pred_offset, connections):
    act_weight = jax.nn.sigmoid(
        jnp.einsum('nd,ndo->no', node_state, nodes_act))
    act_weight = jnp.squeeze(act_weight, axis=-1)
    act_mask = act_weight > act_noise
    act_indices = jnp.nonzero(act_mask, size=MA, fill_value=-1)[0]
    null_act = act_indices == -1
    act_indices = jnp.maximum(act_indices, 0)
    act_vals = jnp.einsum('ek,erk->er', node_state[act_indices],
                          nodes[act_indices])
    stz = node_state.at[act_indices].set(0.0)
    conns = jax.nn.softmax(connections[act_indices], axis=-1)
    top_w, top_i = jax.lax.top_k(conns, KC)
    x = top_w[:, :, None] * act_vals[:, None, :]
    x = jnp.where(null_act[:, None, None], 0.0, x)
    ni = jnp.zeros((N, DM), jnp.float32).at[top_i.reshape(-1)].add(
        x.reshape(-1, DM))

    BN = 128
    grid = N // BN
    ns, nip, loss = pl.pallas_call(
        _final_body,
        grid=(grid,),
        in_specs=[
            pl.BlockSpec((BN, DM), lambda i: (i, 0)),
            pl.BlockSpec((BN, DM), lambda i: (i, 0)),
            pl.BlockSpec((BN, DM), lambda i: (i, 0)),
            pl.BlockSpec((BN, DM, DM), lambda i: (i, 0, 0)),
            pl.BlockSpec((BN, DM), lambda i: (i, 0)),
            pl.BlockSpec((BN, DM), lambda i: (i, 0)),
        ],
        out_specs=[
            pl.BlockSpec((BN, DM), lambda i: (i, 0)),
            pl.BlockSpec((BN, DM), lambda i: (i, 0)),
            pl.BlockSpec((1, 1), lambda i: (0, 0)),
        ],
        out_shape=[
            jax.ShapeDtypeStruct((N, DM), jnp.float32),
            jax.ShapeDtypeStruct((N, DM), jnp.float32),
            jax.ShapeDtypeStruct((1, 1), jnp.float32),
        ],
        scratch_shapes=[pltpu.SMEM((1,), jnp.float32)],
    )(stz, ni, input_pred, pred_nodes, pred_scale, pred_offset)
    return (ns, node_exhaustion, nip, jnp.reshape(loss, ()))


# speedup vs baseline: 1.0683x; 1.0683x over previous
import jax
import jax.numpy as jnp
from jax.experimental import pallas as pl
from jax.experimental.pallas import tpu as pltpu

DM = 64
N = 4096
KC = 32
MA = 256
ALPHA = 0.98
BA = 8
NEG = -3.0e38


def _route_body(idx_ref, conn_hbm, ns_hbm, nodes_hbm, nullmul_ref,
                x_ref, ti_ref, rowbuf, matbuf, vecbuf, sem):
    s = pl.program_id(0)
    for r in range(BA):
        e = idx_ref[s * BA + r]
        pltpu.make_async_copy(conn_hbm.at[e], rowbuf.at[r], sem).start()
        pltpu.make_async_copy(nodes_hbm.at[e], matbuf.at[r], sem).start()
        pltpu.make_async_copy(ns_hbm.at[e], vecbuf.at[r], sem).start()
    for r in range(BA):
        pltpu.make_async_copy(conn_hbm.at[0], rowbuf.at[r], sem).wait()
        pltpu.make_async_copy(nodes_hbm.at[0], matbuf.at[r], sem).wait()
        pltpu.make_async_copy(ns_hbm.at[0], vecbuf.at[r], sem).wait()

    av = jnp.sum(matbuf[...] * vecbuf[...][:, None, :], axis=2)

    row = rowbuf[...]
    m = jnp.max(row, axis=1, keepdims=True)
    ssum = jnp.sum(jnp.exp(row - m), axis=1, keepdims=True)

    iota = jax.lax.broadcasted_iota(jnp.int32, (BA, N), 1)
    r_ = row
    vals = []
    idxs = []
    for _ in range(KC):
        mj = jnp.max(r_, axis=1, keepdims=True)
        ij = jnp.min(jnp.where(r_ == mj, iota, N), axis=1,
                     keepdims=True)
        vals.append(mj)
        idxs.append(ij)
        r_ = jnp.where(iota == ij, NEG, r_)
    w = jnp.exp(jnp.concatenate(vals, axis=1) - m) / ssum
    ti = jnp.concatenate(idxs, axis=1)

    notnull = nullmul_ref[...]
    x_ref[...] = (w[:, :, None] * av[:, None, :]) * notnull[:, :, None]
    ti_ref[...] = ti


def _final_body(stz_ref, ni_ref, ip_ref, pn_ref, ps_ref, po_ref,
                ns_out, nip_out, loss_out, acc_ref):
    i = pl.program_id(0)
    ni = ni_ref[...]
    nip = jnp.zeros_like(ni)
    for d in range(DM):
        nip = nip + ni[:, d:d + 1] * pn_ref[:, d, :]
    diff = ip_ref[...] - ni
    part = jnp.sum(0.5 * diff * diff)

    @pl.when(i == 0)
    def _():
        acc_ref[0] = 0.0

    acc_ref[0] += part

    @pl.when(i == pl.num_programs(0) - 1)
    def _():
        loss_out[...] = jnp.full((1, 1), acc_ref[0] / (N * DM), jnp.float32)

    nip_out[...] = nip
    ns_out[...] = (stz_ref[...] + ni + nip * ps_ref[...] + po_ref[...]) * ALPHA


def kernel(node_state, node_exhaustion, input_pred, act_noise, nodes,
           pred_nodes, nodes_act, pred_scale, pred_offset, connections):
    act_weight = jax.nn.sigmoid(
        jnp.einsum('nd,ndo->no', node_state, nodes_act))
    act_weight = jnp.squeeze(act_weight, axis=-1)
    act_mask = act_weight > act_noise
    act_indices = jnp.nonzero(act_mask, size=MA, fill_value=-1)[0]
    null_act = act_indices == -1
    act_indices = jnp.maximum(act_indices, 0).astype(jnp.int32)
    nullmul = (1.0 - null_act.astype(jnp.float32)).reshape(MA, 1)

    grid = MA // BA
    x, ti = pl.pallas_call(
        _route_body,
        grid_spec=pltpu.PrefetchScalarGridSpec(
            num_scalar_prefetch=1,
            grid=(grid,),
            in_specs=[
                pl.BlockSpec(memory_space=pl.ANY),
                pl.BlockSpec(memory_space=pl.ANY),
                pl.BlockSpec(memory_space=pl.ANY),
                pl.BlockSpec((BA, 1), lambda s, idx: (s, 0)),
            ],
            out_specs=[
                pl.BlockSpec((BA, KC, DM), lambda s, idx: (s, 0, 0)),
                pl.BlockSpec((BA, KC), lambda s, idx: (s, 0)),
            ],
            scratch_shapes=[
                pltpu.VMEM((BA, N), jnp.float32),
                pltpu.VMEM((BA, DM, DM), jnp.float32),
                pltpu.VMEM((BA, DM), jnp.float32),
                pltpu.SemaphoreType.DMA,
            ],
        ),
        out_shape=[
            jax.ShapeDtypeStruct((MA, KC, DM), jnp.float32),
            jax.ShapeDtypeStruct((MA, KC), jnp.int32),
        ],
    )(act_indices, connections, node_state, nodes, nullmul)

    stz = node_state.at[act_indices].set(0.0)
    ni = jnp.zeros((N, DM), jnp.float32).at[ti.reshape(-1)].add(
        x.reshape(-1, DM))

    BN = 128
    ns, nip, loss = pl.pallas_call(
        _final_body,
        grid=(N // BN,),
        in_specs=[
            pl.BlockSpec((BN, DM), lambda i: (i, 0)),
            pl.BlockSpec((BN, DM), lambda i: (i, 0)),
            pl.BlockSpec((BN, DM), lambda i: (i, 0)),
            pl.BlockSpec((BN, DM, DM), lambda i: (i, 0, 0)),
            pl.BlockSpec((BN, DM), lambda i: (i, 0)),
            pl.BlockSpec((BN, DM), lambda i: (i, 0)),
        ],
        out_specs=[
            pl.BlockSpec((BN, DM), lambda i: (i, 0)),
            pl.BlockSpec((BN, DM), lambda i: (i, 0)),
            pl.BlockSpec((1, 1), lambda i: (0, 0)),
        ],
        out_shape=[
            jax.ShapeDtypeStruct((N, DM), jnp.float32),
            jax.ShapeDtypeStruct((N, DM), jnp.float32),
            jax.ShapeDtypeStruct((1, 1), jnp.float32),
        ],
        scratch_shapes=[pltpu.SMEM((1,), jnp.float32)],
    )(stz, ni, input_pred, pred_nodes, pred_scale, pred_offset)
    return (ns, node_exhaustion, nip, jnp.reshape(loss, ()))


# speedup vs baseline: 1.5618x; 1.4620x over previous
import jax
import jax.numpy as jnp
from jax.experimental import pallas as pl
from jax.experimental.pallas import tpu as pltpu

DM = 64
N = 4096
KC = 32
MA = 256
ALPHA = 0.98
BA = 32
NEG = -3.0e38


def _route_body(idx_ref, conn_hbm, ns_hbm, nodes_hbm, nullmul_ref,
                x_ref, ti_ref, rowbuf, matbuf, vecbuf, sem):
    s = pl.program_id(0)
    for r in range(BA):
        e = idx_ref[s * BA + r]
        pltpu.make_async_copy(conn_hbm.at[e], rowbuf.at[r], sem).start()
        pltpu.make_async_copy(nodes_hbm.at[e], matbuf.at[r], sem).start()
        pltpu.make_async_copy(ns_hbm.at[e], vecbuf.at[r], sem).start()
    for r in range(BA):
        pltpu.make_async_copy(conn_hbm.at[0], rowbuf.at[r], sem).wait()
        pltpu.make_async_copy(nodes_hbm.at[0], matbuf.at[r], sem).wait()
        pltpu.make_async_copy(ns_hbm.at[0], vecbuf.at[r], sem).wait()

    av = jnp.sum(matbuf[...] * vecbuf[...][:, None, :], axis=2)

    row = rowbuf[...]
    m = jnp.max(row, axis=1, keepdims=True)
    ssum = jnp.sum(jnp.exp(row - m), axis=1, keepdims=True)

    iota = jax.lax.broadcasted_iota(jnp.int32, (BA, N), 1)
    r_ = row
    vals = []
    idxs = []
    for _ in range(KC):
        mj = jnp.max(r_, axis=1, keepdims=True)
        ij = jnp.min(jnp.where(r_ == mj, iota, N), axis=1,
                     keepdims=True)
        vals.append(mj)
        idxs.append(ij)
        r_ = jnp.where(iota == ij, NEG, r_)
    w = jnp.exp(jnp.concatenate(vals, axis=1) - m) / ssum
    ti = jnp.concatenate(idxs, axis=1)

    notnull = nullmul_ref[...]
    x_ref[...] = (w[:, :, None] * av[:, None, :]) * notnull[:, :, None]
    ti_ref[...] = ti


def _final_body(stz_ref, ni_ref, ip_ref, pn_ref, ps_ref, po_ref,
                ns_out, nip_out, loss_out, acc_ref):
    i = pl.program_id(0)
    ni = ni_ref[...]
    nip = jnp.zeros_like(ni)
    for d in range(DM):
        nip = nip + ni[:, d:d + 1] * pn_ref[:, d, :]
    diff = ip_ref[...] - ni
    part = jnp.sum(0.5 * diff * diff)

    @pl.when(i == 0)
    def _():
        acc_ref[0] = 0.0

    acc_ref[0] += part

    @pl.when(i == pl.num_programs(0) - 1)
    def _():
        loss_out[...] = jnp.full((1, 1), acc_ref[0] / (N * DM), jnp.float32)

    nip_out[...] = nip
    ns_out[...] = (stz_ref[...] + ni + nip * ps_ref[...] + po_ref[...]) * ALPHA


def kernel(node_state, node_exhaustion, input_pred, act_noise, nodes,
           pred_nodes, nodes_act, pred_scale, pred_offset, connections):
    act_weight = jax.nn.sigmoid(
        jnp.einsum('nd,ndo->no', node_state, nodes_act))
    act_weight = jnp.squeeze(act_weight, axis=-1)
    act_mask = act_weight > act_noise
    act_indices = jnp.nonzero(act_mask, size=MA, fill_value=-1)[0]
    null_act = act_indices == -1
    act_indices = jnp.maximum(act_indices, 0).astype(jnp.int32)
    nullmul = (1.0 - null_act.astype(jnp.float32)).reshape(MA, 1)

    grid = MA // BA
    x, ti = pl.pallas_call(
        _route_body,
        grid_spec=pltpu.PrefetchScalarGridSpec(
            num_scalar_prefetch=1,
            grid=(grid,),
            in_specs=[
                pl.BlockSpec(memory_space=pl.ANY),
                pl.BlockSpec(memory_space=pl.ANY),
                pl.BlockSpec(memory_space=pl.ANY),
                pl.BlockSpec((BA, 1), lambda s, idx: (s, 0)),
            ],
            out_specs=[
                pl.BlockSpec((BA, KC, DM), lambda s, idx: (s, 0, 0)),
                pl.BlockSpec((BA, KC), lambda s, idx: (s, 0)),
            ],
            scratch_shapes=[
                pltpu.VMEM((BA, N), jnp.float32),
                pltpu.VMEM((BA, DM, DM), jnp.float32),
                pltpu.VMEM((BA, DM), jnp.float32),
                pltpu.SemaphoreType.DMA,
            ],
        ),
        out_shape=[
            jax.ShapeDtypeStruct((MA, KC, DM), jnp.float32),
            jax.ShapeDtypeStruct((MA, KC), jnp.int32),
        ],
    )(act_indices, connections, node_state, nodes, nullmul)

    stz = node_state.at[act_indices].set(0.0)
    ni = jnp.zeros((N, DM), jnp.float32).at[ti.reshape(-1)].add(
        x.reshape(-1, DM))

    BN = 128
    ns, nip, loss = pl.pallas_call(
        _final_body,
        grid=(N // BN,),
        in_specs=[
            pl.BlockSpec((BN, DM), lambda i: (i, 0)),
            pl.BlockSpec((BN, DM), lambda i: (i, 0)),
            pl.BlockSpec((BN, DM), lambda i: (i, 0)),
            pl.BlockSpec((BN, DM, DM), lambda i: (i, 0, 0)),
            pl.BlockSpec((BN, DM), lambda i: (i, 0)),
            pl.BlockSpec((BN, DM), lambda i: (i, 0)),
        ],
        out_specs=[
            pl.BlockSpec((BN, DM), lambda i: (i, 0)),
            pl.BlockSpec((BN, DM), lambda i: (i, 0)),
            pl.BlockSpec((1, 1), lambda i: (0, 0)),
        ],
        out_shape=[
            jax.ShapeDtypeStruct((N, DM), jnp.float32),
            jax.ShapeDtypeStruct((N, DM), jnp.float32),
            jax.ShapeDtypeStruct((1, 1), jnp.float32),
        ],
        scratch_shapes=[pltpu.SMEM((1,), jnp.float32)],
    )(stz, ni, input_pred, pred_nodes, pred_scale, pred_offset)
    return (ns, node_exhaustion, nip, jnp.reshape(loss, ()))


# speedup vs baseline: 1.6874x; 1.0805x over previous
import jax
import jax.numpy as jnp
from jax.experimental import pallas as pl
from jax.experimental.pallas import tpu as pltpu

DM = 64
N = 4096
KC = 32
MA = 256
ALPHA = 0.98
BA = 64
NEG = -3.0e38


def _route_body(idx_ref, conn_hbm, ns_hbm, nodes_hbm, nullmul_ref,
                x_ref, ti_ref, rowbuf, matbuf, vecbuf, sem):
    s = pl.program_id(0)
    for r in range(BA):
        e = idx_ref[s * BA + r]
        pltpu.make_async_copy(conn_hbm.at[e], rowbuf.at[r], sem).start()
        pltpu.make_async_copy(nodes_hbm.at[e], matbuf.at[r], sem).start()
        pltpu.make_async_copy(ns_hbm.at[e], vecbuf.at[r], sem).start()
    for r in range(BA):
        pltpu.make_async_copy(conn_hbm.at[0], rowbuf.at[r], sem).wait()
        pltpu.make_async_copy(nodes_hbm.at[0], matbuf.at[r], sem).wait()
        pltpu.make_async_copy(ns_hbm.at[0], vecbuf.at[r], sem).wait()

    av = jnp.sum(matbuf[...] * vecbuf[...][:, None, :], axis=2)

    row = rowbuf[...]
    m = jnp.max(row, axis=1, keepdims=True)
    ssum = jnp.sum(jnp.exp(row - m), axis=1, keepdims=True)

    iota = jax.lax.broadcasted_iota(jnp.int32, (BA, N), 1)
    r_ = row
    vals = []
    idxs = []
    for _ in range(KC):
        mj = jnp.max(r_, axis=1, keepdims=True)
        ij = jnp.min(jnp.where(r_ == mj, iota, N), axis=1,
                     keepdims=True)
        vals.append(mj)
        idxs.append(ij)
        r_ = jnp.where(iota == ij, NEG, r_)
    w = jnp.exp(jnp.concatenate(vals, axis=1) - m) / ssum
    ti = jnp.concatenate(idxs, axis=1)

    notnull = nullmul_ref[...]
    x_ref[...] = (w[:, :, None] * av[:, None, :]) * notnull[:, :, None]
    ti_ref[...] = ti


def _final_body(stz_ref, ni_ref, ip_ref, pn_ref, ps_ref, po_ref,
                ns_out, nip_out, loss_out, acc_ref):
    i = pl.program_id(0)
    ni = ni_ref[...]
    nip = jnp.zeros_like(ni)
    for d in range(DM):
        nip = nip + ni[:, d:d + 1] * pn_ref[:, d, :]
    diff = ip_ref[...] - ni
    part = jnp.sum(0.5 * diff * diff)

    @pl.when(i == 0)
    def _():
        acc_ref[0] = 0.0

    acc_ref[0] += part

    @pl.when(i == pl.num_programs(0) - 1)
    def _():
        loss_out[...] = jnp.full((1, 1), acc_ref[0] / (N * DM), jnp.float32)

    nip_out[...] = nip
    ns_out[...] = (stz_ref[...] + ni + nip * ps_ref[...] + po_ref[...]) * ALPHA


def kernel(node_state, node_exhaustion, input_pred, act_noise, nodes,
           pred_nodes, nodes_act, pred_scale, pred_offset, connections):
    act_weight = jax.nn.sigmoid(
        jnp.einsum('nd,ndo->no', node_state, nodes_act))
    act_weight = jnp.squeeze(act_weight, axis=-1)
    act_mask = act_weight > act_noise
    act_indices = jnp.nonzero(act_mask, size=MA, fill_value=-1)[0]
    null_act = act_indices == -1
    act_indices = jnp.maximum(act_indices, 0).astype(jnp.int32)
    nullmul = (1.0 - null_act.astype(jnp.float32)).reshape(MA, 1)

    grid = MA // BA
    x, ti = pl.pallas_call(
        _route_body,
        grid_spec=pltpu.PrefetchScalarGridSpec(
            num_scalar_prefetch=1,
            grid=(grid,),
            in_specs=[
                pl.BlockSpec(memory_space=pl.ANY),
                pl.BlockSpec(memory_space=pl.ANY),
                pl.BlockSpec(memory_space=pl.ANY),
                pl.BlockSpec((BA, 1), lambda s, idx: (s, 0)),
            ],
            out_specs=[
                pl.BlockSpec((BA, KC, DM), lambda s, idx: (s, 0, 0)),
                pl.BlockSpec((BA, KC), lambda s, idx: (s, 0)),
            ],
            scratch_shapes=[
                pltpu.VMEM((BA, N), jnp.float32),
                pltpu.VMEM((BA, DM, DM), jnp.float32),
                pltpu.VMEM((BA, DM), jnp.float32),
                pltpu.SemaphoreType.DMA,
            ],
        ),
        out_shape=[
            jax.ShapeDtypeStruct((MA, KC, DM), jnp.float32),
            jax.ShapeDtypeStruct((MA, KC), jnp.int32),
        ],
    )(act_indices, connections, node_state, nodes, nullmul)

    stz = node_state.at[act_indices].set(0.0)
    ni = jnp.zeros((N, DM), jnp.float32).at[ti.reshape(-1)].add(
        x.reshape(-1, DM))

    BN = 128
    ns, nip, loss = pl.pallas_call(
        _final_body,
        grid=(N // BN,),
        in_specs=[
            pl.BlockSpec((BN, DM), lambda i: (i, 0)),
            pl.BlockSpec((BN, DM), lambda i: (i, 0)),
            pl.BlockSpec((BN, DM), lambda i: (i, 0)),
            pl.BlockSpec((BN, DM, DM), lambda i: (i, 0, 0)),
            pl.BlockSpec((BN, DM), lambda i: (i, 0)),
            pl.BlockSpec((BN, DM), lambda i: (i, 0)),
        ],
        out_specs=[
            pl.BlockSpec((BN, DM), lambda i: (i, 0)),
            pl.BlockSpec((BN, DM), lambda i: (i, 0)),
            pl.BlockSpec((1, 1), lambda i: (0, 0)),
        ],
        out_shape=[
            jax.ShapeDtypeStruct((N, DM), jnp.float32),
            jax.ShapeDtypeStruct((N, DM), jnp.float32),
            jax.ShapeDtypeStruct((1, 1), jnp.float32),
        ],
        scratch_shapes=[pltpu.SMEM((1,), jnp.float32)],
    )(stz, ni, input_pred, pred_nodes, pred_scale, pred_offset)
    return (ns, node_exhaustion, nip, jnp.reshape(loss, ()))
